# baseline (speedup 1.0000x reference)
import functools

import jax
import jax.numpy as jnp
from jax import lax
from jax.experimental import pallas as pl
from jax.experimental.pallas import tpu as pltpu
from jax.experimental.pallas import tpu_sc as plsc

N = 10000
E = 320000
D = 128

NW = 32
CPW = D // NW
NP = 10240
CHUNK = 3200
NCHUNK = E // CHUNK
GPC = CHUNK // 16
DEG_R = 10240


def _sc_body(row_h, col_h, ew_h, featsT_h, accT_o, deg_o,
             ridx, cidx, wv, feats_l, acc_l, deg_l, fsem, csem):
    cid = lax.axis_index("c")
    sid = lax.axis_index("s")
    wid = cid * 16 + sid

    fslice = pl.ds(wid * CPW * NP, CPW * NP)
    pltpu.async_copy(featsT_h.at[fslice], feats_l, fsem)
    pltpu.async_copy(row_h.at[pl.ds(0, CHUNK)], ridx.at[0], csem.at[0])
    pltpu.async_copy(col_h.at[pl.ds(0, CHUNK)], cidx.at[0], csem.at[0])
    pltpu.async_copy(ew_h.at[pl.ds(0, CHUNK)], wv.at[0], csem.at[0])

    zero16 = jnp.zeros((16,), jnp.float32)

    @pl.loop(0, CPW * NP // 16)
    def _zero_acc(i):
        acc_l[pl.ds(i * 16, 16)] = zero16

    @pl.loop(0, DEG_R // 16)
    def _zero_deg(i):
        deg_l[pl.ds(i * 16, 16)] = zero16

    pltpu.make_async_copy(featsT_h.at[fslice], feats_l, fsem).wait()

    base_c = [jnp.full((16,), c * NP, jnp.int32) for c in range(CPW)]

    @pl.loop(0, NCHUNK)
    def _chunk(c):
        kb = c % 2
        kbn = 1 - kb

        @pl.when(c + 1 < NCHUNK)
        def _prefetch():
            nxt = pl.ds((c + 1) * CHUNK, CHUNK)
            pltpu.async_copy(row_h.at[nxt], ridx.at[kbn], csem.at[kbn])
            pltpu.async_copy(col_h.at[nxt], cidx.at[kbn], csem.at[kbn])
            pltpu.async_copy(ew_h.at[nxt], wv.at[kbn], csem.at[kbn])

        cur = pl.ds(c * CHUNK, CHUNK)
        pltpu.make_async_copy(row_h.at[cur], ridx.at[kb], csem.at[kb]).wait()
        pltpu.make_async_copy(col_h.at[cur], cidx.at[kb], csem.at[kb]).wait()
        pltpu.make_async_copy(ew_h.at[cur], wv.at[kb], csem.at[kb]).wait()

        @plsc.parallel_loop(0, GPC, unroll=4)
        def _group(g):
            sl = pl.ds(g * 16, 16)
            r16 = ridx[kb, sl]
            c16 = cidx[kb, sl]
            w16 = wv[kb, sl]
            for cc in range(CPW):
                v = plsc.load_gather(feats_l, [c16 + base_c[cc]])
                plsc.addupdate_scatter(acc_l, [r16 + base_c[cc]], v * w16)
            plsc.addupdate_scatter(deg_l, [r16], w16)

    pltpu.sync_copy(acc_l, accT_o.at[fslice])

    @pl.when(wid == 0)
    def _write_deg():
        pltpu.sync_copy(deg_l, deg_o)


_sc_agg = functools.partial(
    pl.kernel,
    out_type=(jax.ShapeDtypeStruct((NW * CPW * NP,), jnp.float32),
              jax.ShapeDtypeStruct((DEG_R,), jnp.float32)),
    mesh=plsc.VectorSubcoreMesh(core_axis_name="c", subcore_axis_name="s"),
    compiler_params=pltpu.CompilerParams(needs_layout_passes=False),
    scratch_types=[
        pltpu.VMEM((2, CHUNK), jnp.int32),
        pltpu.VMEM((2, CHUNK), jnp.int32),
        pltpu.VMEM((2, CHUNK), jnp.float32),
        pltpu.VMEM((CPW * NP,), jnp.float32),
        pltpu.VMEM((CPW * NP,), jnp.float32),
        pltpu.VMEM((DEG_R,), jnp.float32),
        pltpu.SemaphoreType.DMA,
        pltpu.SemaphoreType.DMA((2,)),
    ],
)(_sc_body)


BLK = 2048


def _tc_body(aT, dT, f, w, bb, o):
    agg = aT[...] * (1.0 / dT[...])
    h = lax.dot_general(agg, w[...], (((0,), (1,)), ((), ())),
                        preferred_element_type=jnp.float32)
    o[...] = f[...] + jnp.maximum(h + bb[...], 0.0)


def _tc_post(accT, deg, feats, W, b2):
    return pl.pallas_call(
        _tc_body,
        grid=(pl.cdiv(N, BLK),),
        in_specs=[
            pl.BlockSpec((D, BLK), lambda i: (0, i)),
            pl.BlockSpec((1, BLK), lambda i: (0, i)),
            pl.BlockSpec((BLK, D), lambda i: (i, 0)),
            pl.BlockSpec((D, D), lambda i: (0, 0)),
            pl.BlockSpec((1, D), lambda i: (0, 0)),
        ],
        out_specs=pl.BlockSpec((BLK, D), lambda i: (i, 0)),
        out_shape=jax.ShapeDtypeStruct((N, D), jnp.float32),
    )(accT, deg, feats, W, b2)


@jax.jit
def kernel(edge_index, edge_weight, feats, W, b):
    row = edge_index[0].astype(jnp.int32)
    col = edge_index[1].astype(jnp.int32)
    ew = edge_weight.astype(jnp.float32)
    featsT = jnp.pad(feats.T, ((0, 0), (0, NP - N))).reshape(-1)

    accT, deg = _sc_agg(row, col, ew, featsT)
    return _tc_post(accT.reshape(D, NP), deg[:NP].reshape(1, NP),
                    feats, W, b.reshape(1, D))


# speedup vs baseline: 1.0773x; 1.0773x over previous
import functools

import jax
import jax.numpy as jnp
from jax import lax
from jax.experimental import pallas as pl
from jax.experimental.pallas import tpu as pltpu
from jax.experimental.pallas import tpu_sc as plsc

N = 10000
E = 320000
D = 128

NW = 32
CPW = D // NW
NP = 10240
CHUNK = 3200
NCHUNK = E // CHUNK
GPC = CHUNK // 16
DEG_R = 384


def _sc_body(row_h, col_h, ew_h, featsT_h, accT_o, deg_o,
             ridx, cidx, wv, feats_l, acc_l, deg_l, fsem, csem):
    cid = lax.axis_index("c")
    sid = lax.axis_index("s")
    wid = cid * 16 + sid

    fslice = pl.ds(wid * CPW * NP, CPW * NP)
    pltpu.async_copy(featsT_h.at[fslice], feats_l, fsem)
    pltpu.async_copy(row_h.at[pl.ds(0, CHUNK)], ridx.at[0], csem.at[0])
    pltpu.async_copy(col_h.at[pl.ds(0, CHUNK)], cidx.at[0], csem.at[0])
    pltpu.async_copy(ew_h.at[pl.ds(0, CHUNK)], wv.at[0], csem.at[0])

    zero16 = jnp.zeros((16,), jnp.float32)

    @pl.loop(0, CPW * NP // 16)
    def _zero_acc(i):
        acc_l[pl.ds(i * 16, 16)] = zero16

    @pl.loop(0, DEG_R // 16)
    def _zero_deg(i):
        deg_l[pl.ds(i * 16, 16)] = zero16

    pltpu.make_async_copy(featsT_h.at[fslice], feats_l, fsem).wait()

    base_c = [jnp.full((16,), c * NP, jnp.int32) for c in range(CPW)]
    lo16 = jnp.full((16,), wid * DEG_R, jnp.int32)
    hi16 = jnp.full((16,), wid * DEG_R + DEG_R, jnp.int32)

    @pl.loop(0, NCHUNK)
    def _chunk(c):
        kb = c % 2
        kbn = 1 - kb

        @pl.when(c + 1 < NCHUNK)
        def _prefetch():
            nxt = pl.ds((c + 1) * CHUNK, CHUNK)
            pltpu.async_copy(row_h.at[nxt], ridx.at[kbn], csem.at[kbn])
            pltpu.async_copy(col_h.at[nxt], cidx.at[kbn], csem.at[kbn])
            pltpu.async_copy(ew_h.at[nxt], wv.at[kbn], csem.at[kbn])

        cur = pl.ds(c * CHUNK, CHUNK)
        pltpu.make_async_copy(row_h.at[cur], ridx.at[kb], csem.at[kb]).wait()
        pltpu.make_async_copy(col_h.at[cur], cidx.at[kb], csem.at[kb]).wait()
        pltpu.make_async_copy(ew_h.at[cur], wv.at[kb], csem.at[kb]).wait()

        @plsc.parallel_loop(0, GPC, unroll=4)
        def _group(g):
            sl = pl.ds(g * 16, 16)
            r16 = ridx[kb, sl]
            c16 = cidx[kb, sl]
            w16 = wv[kb, sl]
            for cc in range(CPW):
                v = plsc.load_gather(feats_l, [c16 + base_c[cc]])
                plsc.addupdate_scatter(acc_l, [r16 + base_c[cc]], v * w16)
            mask = (r16 >= lo16) & (r16 < hi16)
            plsc.addupdate_scatter(deg_l, [r16 - lo16], w16, mask=mask)

    pltpu.sync_copy(acc_l, accT_o.at[fslice])
    pltpu.sync_copy(deg_l, deg_o.at[pl.ds(wid * DEG_R, DEG_R)])


_sc_agg = functools.partial(
    pl.kernel,
    out_type=(jax.ShapeDtypeStruct((NW * CPW * NP,), jnp.float32),
              jax.ShapeDtypeStruct((NW * DEG_R,), jnp.float32)),
    mesh=plsc.VectorSubcoreMesh(core_axis_name="c", subcore_axis_name="s"),
    compiler_params=pltpu.CompilerParams(needs_layout_passes=False),
    scratch_types=[
        pltpu.VMEM((2, CHUNK), jnp.int32),
        pltpu.VMEM((2, CHUNK), jnp.int32),
        pltpu.VMEM((2, CHUNK), jnp.float32),
        pltpu.VMEM((CPW * NP,), jnp.float32),
        pltpu.VMEM((CPW * NP,), jnp.float32),
        pltpu.VMEM((DEG_R,), jnp.float32),
        pltpu.SemaphoreType.DMA,
        pltpu.SemaphoreType.DMA((2,)),
    ],
)(_sc_body)


BLK = 2048


def _tc_body(aT, dT, f, w, bb, o):
    agg = aT[...] * (1.0 / dT[...])
    h = lax.dot_general(agg, w[...], (((0,), (1,)), ((), ())),
                        preferred_element_type=jnp.float32)
    o[...] = f[...] + jnp.maximum(h + bb[...], 0.0)


def _tc_post(accT, deg, feats, W, b2):
    return pl.pallas_call(
        _tc_body,
        grid=(pl.cdiv(N, BLK),),
        in_specs=[
            pl.BlockSpec((D, BLK), lambda i: (0, i)),
            pl.BlockSpec((1, BLK), lambda i: (0, i)),
            pl.BlockSpec((BLK, D), lambda i: (i, 0)),
            pl.BlockSpec((D, D), lambda i: (0, 0)),
            pl.BlockSpec((1, D), lambda i: (0, 0)),
        ],
        out_specs=pl.BlockSpec((BLK, D), lambda i: (i, 0)),
        out_shape=jax.ShapeDtypeStruct((N, D), jnp.float32),
    )(accT, deg, feats, W, b2)


@jax.jit
def kernel(edge_index, edge_weight, feats, W, b):
    row = edge_index[0].astype(jnp.int32)
    col = edge_index[1].astype(jnp.int32)
    ew = edge_weight.astype(jnp.float32)
    featsT = jnp.pad(feats.T, ((0, 0), (0, NP - N))).reshape(-1)

    accT, deg = _sc_agg(row, col, ew, featsT)
    return _tc_post(accT.reshape(D, NP), deg[:NP].reshape(1, NP),
                    feats, W, b.reshape(1, D))
